# baseline (speedup 1.0000x reference)
import functools

import jax
import jax.numpy as jnp
from jax import lax
from jax.experimental import pallas as pl
from jax.experimental.pallas import tpu as pltpu
from jax.experimental.pallas import tpu_sc as plsc

_NC = 2
_NS = 16
_CHUNK = 32
_NBUF = 4
_LANE = 128


@functools.lru_cache(maxsize=None)
def _make_gather(M, n_loc, nope_dim, rope_dim):
    NW = _NC * _NS
    per_w = n_loc // NW
    n_ch = per_w // _CHUNK
    assert n_ch % _NBUF == 0 and n_ch >= 2 * _NBUF
    n_grp = n_ch // _NBUF
    tail_off = _LANE - rope_dim
    mesh = plsc.VectorSubcoreMesh(core_axis_name="c", subcore_axis_name="s")

    @functools.partial(
        pl.kernel,
        mesh=mesh,
        out_type=(
            jax.ShapeDtypeStruct((n_loc, nope_dim), jnp.float32),
            jax.ShapeDtypeStruct((n_loc, rope_dim), jnp.float32),
        ),
        scratch_types=[
            pltpu.VMEM((per_w,), jnp.int32),
            [pltpu.VMEM((_CHUNK, nope_dim), jnp.float32) for _ in range(_NBUF)],
            [pltpu.VMEM((_CHUNK, _LANE), jnp.float32) for _ in range(_NBUF)],
            [pltpu.VMEM((_CHUNK, rope_dim), jnp.float32) for _ in range(_NBUF)],
            [pltpu.SemaphoreType.DMA for _ in range(_NBUF)],
            [pltpu.SemaphoreType.DMA for _ in range(_NBUF)],
        ],
    )
    def gather_kernel(kv_hbm, tail_hbm, loc_hbm, nope_hbm, rope_hbm,
                      idx_v, nbufs, rbufs, rbuf2s, gsems, wsems):
        wid = lax.axis_index("s") * _NC + lax.axis_index("c")
        base = wid * per_w

        def gathers(j, b):
            idx = idx_v.at[pl.ds(j * _CHUNK, _CHUNK)]
            return (
                pltpu.make_async_copy(
                    kv_hbm.at[idx, pl.ds(0, nope_dim)], nbufs[b], gsems[b]),
                pltpu.make_async_copy(tail_hbm.at[idx], rbufs[b], gsems[b]),
            )

        def writes(j, b):
            row0 = base + j * _CHUNK
            return (
                pltpu.make_async_copy(
                    nbufs[b], nope_hbm.at[pl.ds(row0, _CHUNK), pl.ds(0, nope_dim)],
                    wsems[b]),
                pltpu.make_async_copy(
                    rbuf2s[b],
                    rope_hbm.at[pl.ds(row0, _CHUNK), pl.ds(0, rope_dim)],
                    wsems[b]),
            )

        def stage_rope(b):
            for r in range(_CHUNK):
                for c in range(rope_dim // 16):
                    rbuf2s[b][r, pl.ds(c * 16, 16)] = (
                        rbufs[b][r, pl.ds(tail_off + c * 16, 16)])

        pltpu.sync_copy(loc_hbm.at[pl.ds(base, per_w)], idx_v)
        for j in range(_NBUF - 1):
            for gth in gathers(j, j):
                gth.start()

        def body(g, _):
            for b in range(_NBUF):
                j = g * _NBUF + b
                for gth in gathers(j, b):
                    gth.wait()
                stage_rope(b)
                for w in writes(j, b):
                    w.start()
                bp = (b - 1) % _NBUF

                def drain_prev():
                    for w in writes(j - 1, bp):
                        w.wait()

                def refill():
                    drain_prev()
                    for gth in gathers(j + _NBUF - 1, bp):
                        gth.start()

                if b == 0:
                    pl.when(g >= 1)(drain_prev)
                    for gth in gathers(j + _NBUF - 1, bp):
                        gth.start()
                else:
                    pl.when(g < n_grp - 1)(refill)
            return ()

        lax.fori_loop(0, n_grp, body, (), unroll=False)
        for j in range(n_ch - _NBUF, n_ch):
            for w in writes(j, j % _NBUF):
                w.wait()

    return gather_kernel


def kernel(kv_buffer, loc, cache_k_nope, cache_k_rope):
    M, D = kv_buffer.shape
    n_loc = loc.shape[0]
    nope_dim = cache_k_nope.shape[-1]
    rope_dim = cache_k_rope.shape[-1]
    nope_src = lax.slice(kv_buffer, (0, 0), (M, nope_dim))
    tail = lax.slice(kv_buffer, (0, D - _LANE), (M, D))
    fn = _make_gather(M, n_loc, nope_dim, rope_dim)
    return fn(nope_src, tail, loc)


# speedup vs baseline: 1.3157x; 1.3157x over previous
import functools

import jax
import jax.numpy as jnp
from jax import lax
from jax.experimental import pallas as pl
from jax.experimental.pallas import tpu as pltpu
from jax.experimental.pallas import tpu_sc as plsc

_NC = 2
_NS = 16
_CHUNK = 32
_NBUF = 4
_LANE = 128


@functools.lru_cache(maxsize=None)
def _make_gather(M, n_loc, nope_dim, rope_dim):
    NW = _NC * _NS
    per_w = n_loc // NW
    n_ch = per_w // _CHUNK
    assert n_ch % _NBUF == 0 and n_ch >= 2 * _NBUF
    n_grp = n_ch // _NBUF
    tail_off = _LANE - rope_dim
    mesh = plsc.VectorSubcoreMesh(core_axis_name="c", subcore_axis_name="s")

    @functools.partial(
        pl.kernel,
        mesh=mesh,
        out_type=(
            jax.ShapeDtypeStruct((n_loc, nope_dim), jnp.float32),
            jax.ShapeDtypeStruct((n_loc, rope_dim), jnp.float32),
        ),
        scratch_types=[
            pltpu.VMEM((per_w,), jnp.int32),
            [pltpu.VMEM((_CHUNK, nope_dim), jnp.float32) for _ in range(_NBUF)],
            [pltpu.VMEM((_CHUNK, _LANE), jnp.float32) for _ in range(_NBUF)],
            [pltpu.VMEM((_CHUNK, rope_dim), jnp.float32) for _ in range(_NBUF)],
            [pltpu.SemaphoreType.DMA for _ in range(_NBUF)],
            [pltpu.SemaphoreType.DMA for _ in range(_NBUF)],
        ],
    )
    def gather_kernel(kv_hbm, tail_hbm, loc_hbm, nope_hbm, rope_hbm,
                      idx_v, nbufs, rbufs, rbuf2s, gsems, wsems):
        wid = lax.axis_index("s") * _NC + lax.axis_index("c")
        base = wid * per_w

        def gathers(j, b):
            idx = idx_v.at[pl.ds(j * _CHUNK, _CHUNK)]
            return (
                pltpu.make_async_copy(
                    kv_hbm.at[idx, pl.ds(0, nope_dim)], nbufs[b], gsems[b]),
                pltpu.make_async_copy(tail_hbm.at[idx], rbufs[b], gsems[b]),
            )

        def writes(j, b):
            row0 = base + j * _CHUNK
            return (
                pltpu.make_async_copy(
                    nbufs[b], nope_hbm.at[pl.ds(row0, _CHUNK), pl.ds(0, nope_dim)],
                    wsems[b]),
                pltpu.make_async_copy(
                    rbuf2s[b],
                    rope_hbm.at[pl.ds(row0, _CHUNK), pl.ds(0, rope_dim)],
                    wsems[b]),
            )

        def stage_rope(b):
            for r in range(_CHUNK):
                for c in range(rope_dim // 16):
                    rbuf2s[b][r, pl.ds(c * 16, 16)] = (
                        rbufs[b][r, pl.ds(tail_off + c * 16, 16)])

        pltpu.sync_copy(loc_hbm.at[pl.ds(base, per_w)], idx_v)
        for j in range(_NBUF - 1):
            for gth in gathers(j, j):
                gth.start()

        def body(g, _):
            for b in range(_NBUF):
                j = g * _NBUF + b
                for gth in gathers(j, b):
                    gth.wait()
                stage_rope(b)
                for w in writes(j, b):
                    w.start()
                bp = (b - 1) % _NBUF

                def drain_prev():
                    for w in writes(j - 1, bp):
                        w.wait()

                def refill():
                    drain_prev()
                    for gth in gathers(j + _NBUF - 1, bp):
                        gth.start()

                if b == 0:
                    pl.when(g >= 1)(drain_prev)
                    for gth in gathers(j + _NBUF - 1, bp):
                        gth.start()
                else:
                    pl.when(g < n_grp - 1)(refill)
            return ()

        lax.fori_loop(0, n_grp, body, (), unroll=False)
        for j in range(n_ch - _NBUF, n_ch):
            for w in writes(j, j % _NBUF):
                w.wait()

    return gather_kernel


def kernel(kv_buffer, loc, cache_k_nope, cache_k_rope):
    M, D = kv_buffer.shape
    n_loc = loc.shape[0]
    nope_dim = cache_k_nope.shape[-1]
    rope_dim = cache_k_rope.shape[-1]
    tail = lax.slice(kv_buffer, (0, D - _LANE), (M, D))
    fn = _make_gather(M, n_loc, nope_dim, rope_dim)
    return fn(kv_buffer, tail, loc)
